# Initial kernel scaffold; baseline (speedup 1.0000x reference)
import jax
import jax.numpy as jnp
from jax.experimental import pallas as pl


def kernel(query_points, features, feature_indexs):
    raise NotImplementedError("write your pallas kernel here")


# speedup vs baseline: 3.6011x; 3.6011x over previous
import functools

import jax
import jax.numpy as jnp
from jax import lax
from jax.experimental import pallas as pl
from jax.experimental.pallas import tpu as pltpu
from jax.experimental.pallas import tpu_sc as plsc

FEATURE_DIM = 8
LEAF = 0.01
NLEV = 4
BUF = 10_000_000
NFEAT = 2_000_000
NQ = 262_144

P0M = 73856093 % BUF
P1M = 19349669 % BUF
P2M = 83492791 % BUF

NC, NS = 2, 16
NW = NC * NS
BP = NQ // NW
TILE = 512
NT = BP // TILE
INVB = float(1.0 / BUF)


def _modfix(s):
    return jnp.where(s >= BUF, s - BUF, s)


def _mod_mul(o, m):
    t = o * m
    q = (t.astype(jnp.float32) * INVB).astype(jnp.int32)
    r = t - q * BUF
    r = jnp.where(r < 0, r + BUF, r)
    return jnp.where(r >= BUF, r - BUF, r)


def _vg(v, idx):
    dn = lax.GatherDimensionNumbers(
        offset_dims=(), collapsed_slice_dims=(0,), start_index_map=(0,))
    return lax.gather(v, idx[:, None], dn, (1,),
                      mode=lax.GatherScatterMode.PROMISE_IN_BOUNDS)


def _body(q_hbm, it0, it1, it2, it3, ftab_hbm, out_hbm,
          qv, xv, yv, zv, kbuf, vals, fidx, mbuf, frows, outbuf,
          sem_q, sem_i, sem_f):
    wid = (lax.axis_index("s").astype(jnp.int32) * NC
           + lax.axis_index("c").astype(jnp.int32))
    base = wid * BP
    pltpu.async_copy(q_hbm.at[pl.ds(base, BP)], qv, sem_q).wait()

    _i0 = jnp.int32(0)
    _i8 = jnp.int32(TILE // 16)
    i_refs = [it0, it1, it2, it3]
    f_refs = [ftab_hbm.at[jnp.int32(l)] for l in range(NLEV)]

    lane = lax.iota(jnp.int32, 16)
    half = jnp.where(lane >= 8, jnp.int32(1), jnp.int32(0))
    flane = lane - half * 8
    ax0 = lane * 0

    def tile_body(t, _):
        def tr_body(i, carry):
            pidx = t * TILE + i * 16 + lane
            sl = pl.ds(i * 16, 16)
            xv[sl] = plsc.load_gather(qv, [pidx, ax0])
            yv[sl] = plsc.load_gather(qv, [pidx, ax0 + 1])
            zv[sl] = plsc.load_gather(qv, [pidx, ax0 + 2])
            return carry
        lax.fori_loop(_i0, _i8, tr_body, _i0)

        for l in range(NLEV):
            res = jnp.float32(LEAF * (2.0 ** l))
            _it = i_refs[l]
            _ft = f_refs[l]

            def keys_body(qc, carry, _res=res):
                sl = pl.ds(qc * 16, 16)
                cx = xv[sl] / _res
                cy = yv[sl] / _res
                cz = zv[sl] / _res
                ox = cx.astype(jnp.int32)
                oy = cy.astype(jnp.int32)
                oz = cz.astype(jnp.int32)
                px = [_mod_mul(ox, P0M), None]
                px[1] = _modfix(px[0] + P0M)
                py = [_mod_mul(oy, P1M), None]
                py[1] = _modfix(py[0] + P1M)
                pz = [_mod_mul(oz, P2M), None]
                pz[1] = _modfix(pz[0] + P2M)
                for a in range(2):
                    for b in range(2):
                        sxy = _modfix(px[a] + py[b])
                        for g in range(2):
                            c = a * 4 + b * 2 + g
                            kbuf[pl.ds(c * TILE + qc * 16, 16)] = _modfix(sxy + pz[g])
                return carry
            lax.fori_loop(_i0, _i8, keys_body, _i0)

            pltpu.async_copy(_it.at[kbuf], vals, sem_i).wait()

            def proc_body(qc, carry):
                sl = pl.ds(qc * 16, 16)
                lows = [vals[pl.ds(c * TILE + qc * 16, 16)] for c in range(8)]
                mn = lows[0]
                for c in range(1, 8):
                    mn = jnp.minimum(mn, lows[c])
                mbuf[sl] = jnp.where(mn > -1, jnp.float32(1.0), jnp.float32(0.0))
                for c in range(8):
                    fidx[pl.ds(c * TILE + qc * 16, 16)] = jnp.maximum(lows[c], 0)
                return carry
            lax.fori_loop(_i0, _i8, proc_body, _i0)

            pltpu.async_copy(_ft.at[fidx], frows, sem_f).wait()

            def fma_body(qc, carry, _res=res, _l=l):
                sl = pl.ds(qc * 16, 16)
                cx = xv[sl] / _res
                cy = yv[sl] / _res
                cz = zv[sl] / _res
                tx = cx - cx.astype(jnp.int32).astype(jnp.float32)
                ty = cy - cy.astype(jnp.int32).astype(jnp.float32)
                tz = cz - cz.astype(jnp.int32).astype(jnp.float32)
                mk = mbuf[sl]

                def pair_body(j, c2):
                    rep = 2 * j + half
                    txr = _vg(tx, rep)
                    tyr = _vg(ty, rep)
                    tzr = _vg(tz, rep)
                    mr = _vg(mk, rep)
                    ixr = 1.0 - txr
                    iyr = 1.0 - tyr
                    izr = (1.0 - tzr) * mr
                    tzr2 = tzr * mr
                    pvec = qc * 16 + 2 * j + half
                    f = []
                    for c in range(8):
                        f.append(plsc.load_gather(frows, [c * TILE + pvec, flane]))
                    h00 = f[0] * izr + f[1] * tzr2
                    h01 = f[2] * izr + f[3] * tzr2
                    h10 = f[4] * izr + f[5] * tzr2
                    h11 = f[6] * izr + f[7] * tzr2
                    u0 = h00 * iyr + h01 * tyr
                    u1 = h10 * iyr + h11 * tyr
                    sres = u0 * ixr + u1 * txr
                    osl = pl.ds(qc * 128 + j * 16, 16)
                    if _l == 0:
                        outbuf[osl] = sres
                    else:
                        outbuf[osl] = outbuf[osl] + sres
                    return c2
                lax.fori_loop(_i0, jnp.int32(8), pair_body, _i0)
                return carry
            lax.fori_loop(_i0, _i8, fma_body, _i0)

        pltpu.sync_copy(
            outbuf,
            out_hbm.at[pl.ds(base * FEATURE_DIM + t * (TILE * FEATURE_DIM),
                             TILE * FEATURE_DIM)])
        return _
    lax.fori_loop(_i0, jnp.int32(NT), tile_body, _i0)


def _make_kernel():
    return functools.partial(
        pl.kernel,
        out_type=jax.ShapeDtypeStruct((NQ * FEATURE_DIM,), jnp.float32),
        mesh=plsc.VectorSubcoreMesh(core_axis_name="c", subcore_axis_name="s"),
        compiler_params=pltpu.CompilerParams(
            use_tc_tiling_on_sc=False, needs_layout_passes=False),
        scratch_types=[
            pltpu.VMEM((BP, 3), jnp.float32),
            pltpu.VMEM((TILE,), jnp.float32),
            pltpu.VMEM((TILE,), jnp.float32),
            pltpu.VMEM((TILE,), jnp.float32),
            pltpu.VMEM((8 * TILE,), jnp.int32),
            pltpu.VMEM((8 * TILE,), jnp.int32),
            pltpu.VMEM((8 * TILE,), jnp.int32),
            pltpu.VMEM((TILE,), jnp.float32),
            pltpu.VMEM((8 * TILE, FEATURE_DIM), jnp.float32),
            pltpu.VMEM((TILE * FEATURE_DIM,), jnp.float32),
            pltpu.SemaphoreType.DMA,
            pltpu.SemaphoreType.DMA,
            pltpu.SemaphoreType.DMA,
        ],
    )(_body)


def kernel(query_points, features, feature_indexs):
    its = [feature_indexs[l].astype(jnp.int32) for l in range(NLEV)]
    out = _make_kernel()(query_points, *its, features)
    return out.reshape(NQ, FEATURE_DIM)
